# Initial kernel scaffold; baseline (speedup 1.0000x reference)
import jax
import jax.numpy as jnp
from jax.experimental import pallas as pl


def kernel(x, Wl1, bl1, Wr1, Wl2, bl2, Wr2, Wl3, bl3, Wr3, W1, b1, W2, b2, Wmu, bmu, action_scale, action_bias, ei_src, ei_dst):
    raise NotImplementedError("write your pallas kernel here")


# speedup vs baseline: 4.9492x; 4.9492x over previous
import jax
import jax.numpy as jnp
from jax.experimental import pallas as pl

B = 16384
N = 49
TILE = 512


def _fused_net_kernel(x_ref, k1_ref, c1_ref, k2_ref, c2_ref, k3_ref, c3_ref,
                      w1_ref, b1_ref, w2_ref, b2_ref, wmu_ref, cmu_ref,
                      scale_ref, bias_ref, out_ref):
    x = x_ref[...]
    h = jax.nn.relu(jnp.dot(x, k1_ref[...], preferred_element_type=jnp.float32)
                    + c1_ref[...])
    h = jax.nn.relu(jnp.dot(h, k2_ref[...], preferred_element_type=jnp.float32)
                    + c2_ref[...])
    h = jnp.dot(h, k3_ref[...], preferred_element_type=jnp.float32) + c3_ref[...]
    h = jax.nn.relu(jnp.dot(h, w1_ref[...], preferred_element_type=jnp.float32)
                    + b1_ref[...])
    h = jax.nn.relu(jnp.dot(h, w2_ref[...], preferred_element_type=jnp.float32)
                    + b2_ref[...])
    mu = jnp.tanh(jnp.dot(h, wmu_ref[...], preferred_element_type=jnp.float32)
                  + cmu_ref[...])
    out_ref[...] = mu * scale_ref[...] + bias_ref[...]


def _fold_sage(Amean, Wl, bl, Wr):
    ci, co = Wl.shape
    eye = jnp.eye(N, dtype=jnp.float32)
    K = (Amean.T[:, None, :, None] * Wl[None, :, None, :]
         + eye[:, None, :, None] * Wr[None, :, None, :])
    K = K.reshape(N * ci, N * co)
    c = jnp.tile(bl, N)
    return K, c


def kernel(x, Wl1, bl1, Wr1, Wl2, bl2, Wr2, Wl3, bl3, Wr3, W1, b1, W2, b2,
           Wmu, bmu, action_scale, action_bias, ei_src, ei_dst):
    A = jnp.zeros((N, N), jnp.float32).at[ei_dst, ei_src].add(1.0)
    deg = jnp.zeros((N,), jnp.float32).at[ei_dst].add(1.0)
    Amean = A / jnp.clip(deg, 1.0)[:, None]

    K1, c1 = _fold_sage(Amean, Wl1, bl1, Wr1)
    K2, c2 = _fold_sage(Amean, Wl2, bl2, Wr2)
    K3, c3 = _fold_sage(Amean, Wl3, bl3, Wr3)

    xb = x.reshape(B, N)

    full = lambda *s: pl.BlockSpec(s, lambda i: (0,) * len(s))
    row = lambda n: pl.BlockSpec((1, n), lambda i: (0, 0))

    out = pl.pallas_call(
        _fused_net_kernel,
        grid=(B // TILE,),
        in_specs=[
            pl.BlockSpec((TILE, N), lambda i: (i, 0)),
            full(N, 6 * N), row(6 * N),
            full(6 * N, 6 * N), row(6 * N),
            full(6 * N, 12 * N), row(12 * N),
            full(12 * N, 128), row(128),
            full(128, 128), row(128),
            full(128, 8), row(8),
            row(8), row(8),
        ],
        out_specs=pl.BlockSpec((TILE, 8), lambda i: (i, 0)),
        out_shape=jax.ShapeDtypeStruct((B, 8), jnp.float32),
    )(xb, K1, c1.reshape(1, -1), K2, c2.reshape(1, -1), K3, c3.reshape(1, -1),
      W1, b1.reshape(1, -1), W2, b2.reshape(1, -1), Wmu, bmu.reshape(1, -1),
      action_scale.reshape(1, -1), action_bias.reshape(1, -1))
    return out


# speedup vs baseline: 5.8121x; 1.1743x over previous
import jax
import jax.numpy as jnp
from jax.experimental import pallas as pl

B = 16384
N = 49
TILE = 512


def _fused_net_kernel(x_ref, k1_ref, c1_ref, k2_ref, c2_ref, w31_ref, b31_ref,
                      w2_ref, b2_ref, wmu_ref, cmu_ref,
                      scale_ref, bias_ref, out_ref):
    x = x_ref[...]
    h = jax.nn.relu(jnp.dot(x, k1_ref[...], preferred_element_type=jnp.float32)
                    + c1_ref[...])
    h = jax.nn.relu(jnp.dot(h, k2_ref[...], preferred_element_type=jnp.float32)
                    + c2_ref[...])
    h = jax.nn.relu(jnp.dot(h, w31_ref[...], preferred_element_type=jnp.float32)
                    + b31_ref[...])
    h = jax.nn.relu(jnp.dot(h, w2_ref[...], preferred_element_type=jnp.float32)
                    + b2_ref[...])
    mu = jnp.tanh(jnp.dot(h, wmu_ref[...], preferred_element_type=jnp.float32)
                  + cmu_ref[...])
    out_ref[...] = mu * scale_ref[...] + bias_ref[...]


def _fold_sage(Amean, Wl, bl, Wr):
    ci, co = Wl.shape
    eye = jnp.eye(N, dtype=jnp.float32)
    K = (Amean.T[:, None, :, None] * Wl[None, :, None, :]
         + eye[:, None, :, None] * Wr[None, :, None, :])
    K = K.reshape(N * ci, N * co)
    c = jnp.tile(bl, N)
    return K, c


def kernel(x, Wl1, bl1, Wr1, Wl2, bl2, Wr2, Wl3, bl3, Wr3, W1, b1, W2, b2,
           Wmu, bmu, action_scale, action_bias, ei_src, ei_dst):
    A = jnp.zeros((N, N), jnp.float32).at[ei_dst, ei_src].add(1.0)
    deg = jnp.zeros((N,), jnp.float32).at[ei_dst].add(1.0)
    Amean = A / jnp.clip(deg, 1.0)[:, None]

    K1, c1 = _fold_sage(Amean, Wl1, bl1, Wr1)
    K2, c2 = _fold_sage(Amean, Wl2, bl2, Wr2)
    K3, c3 = _fold_sage(Amean, Wl3, bl3, Wr3)
    W31 = K3 @ W1
    b31 = c3 @ W1 + b1

    xb = x.reshape(B, N)

    full = lambda *s: pl.BlockSpec(s, lambda i: (0,) * len(s))
    row = lambda n: pl.BlockSpec((1, n), lambda i: (0, 0))

    out = pl.pallas_call(
        _fused_net_kernel,
        grid=(B // TILE,),
        in_specs=[
            pl.BlockSpec((TILE, N), lambda i: (i, 0)),
            full(N, 6 * N), row(6 * N),
            full(6 * N, 6 * N), row(6 * N),
            full(6 * N, 128), row(128),
            full(128, 128), row(128),
            full(128, 8), row(8),
            row(8), row(8),
        ],
        out_specs=pl.BlockSpec((TILE, 8), lambda i: (i, 0)),
        out_shape=jax.ShapeDtypeStruct((B, 8), jnp.float32),
    )(xb, K1, c1.reshape(1, -1), K2, c2.reshape(1, -1), W31, b31.reshape(1, -1),
      W2, b2.reshape(1, -1), Wmu, bmu.reshape(1, -1),
      action_scale.reshape(1, -1), action_bias.reshape(1, -1))
    return out
